# baseline (speedup 1.0000x reference)
import functools

import jax
import jax.numpy as jnp
from jax import lax
from jax.experimental import pallas as pl
from jax.experimental.pallas import tpu as pltpu
from jax.experimental.pallas import tpu_sc as plsc

B, S, H, I, E, K = 2, 2048, 1024, 512, 8, 2
N = B * S
TM = 256
TMR = 128
TMR_LOG = 7
NT_TOK = N // TM
T_ROWS = (N * K) // TMR + E
R = T_ROWS * TMR

NC, NS = 2, 16
NW = NC * NS
CW = N // NW
SUB = 64
NSUB = CW // SUB


def _dg16(vals, idx):
    return lax.gather(
        vals, idx[:, None],
        dimension_numbers=lax.GatherDimensionNumbers(
            offset_dims=(), collapsed_slice_dims=(0,), start_index_map=(0,)),
        slice_sizes=(1,),
        mode=lax.GatherScatterMode.PROMISE_IN_BOUNDS)


def _k1_body(x_r, gW_r, gb_r, sWg_r, sbg_r, sWu_r, sbu_r, sWd_r, sbd_r,
             out0_r, w0b_r, w1b_r, e0_r, e1_r, r0_r, r1_r, cnt_r):
    i = pl.program_id(0)
    xb = x_r[...]
    hg = jnp.dot(xb, sWg_r[...], preferred_element_type=jnp.float32) + sbg_r[...]
    hu = jnp.dot(xb, sWu_r[...], preferred_element_type=jnp.float32) + sbu_r[...]
    h = jax.nn.silu(hg) * hu
    out0_r[...] = xb + jnp.dot(h, sWd_r[...], preferred_element_type=jnp.float32) + sbd_r[...]
    logits = jnp.dot(xb, gW_r[...], preferred_element_type=jnp.float32) + gb_r[...]
    p = jax.nn.sigmoid(logits)
    iota_e = lax.broadcasted_iota(jnp.int32, (TM, E), 1)
    v1 = jnp.max(p, axis=1, keepdims=True)
    i1 = jnp.min(jnp.where(p == v1, iota_e, E), axis=1, keepdims=True)
    p2 = jnp.where(iota_e == i1, -1.0, p)
    v2 = jnp.max(p2, axis=1, keepdims=True)
    i2 = jnp.min(jnp.where(p2 == v2, iota_e, E), axis=1, keepdims=True)
    ssum = v1 + v2
    w0b_r[...] = jnp.broadcast_to(v1 / ssum, (TM, 16))
    w1b_r[...] = jnp.broadcast_to(v2 / ssum, (TM, 16))
    e0_r[...] = i1
    e1_r[...] = i2
    @pl.when(i == 0)
    def _():
        cnt_r[...] = jnp.zeros((1, 16), jnp.int32)
    base = cnt_r[:, :E]
    oh1 = (iota_e == i1).astype(jnp.float32)
    oh2 = (iota_e == i2).astype(jnp.float32)
    r_io = lax.broadcasted_iota(jnp.int32, (TM, TM), 0)
    c_io = lax.broadcasted_iota(jnp.int32, (TM, TM), 1)
    lt = (r_io > c_io).astype(jnp.float32)
    c1 = jnp.dot(lt, oh1, preferred_element_type=jnp.float32)
    c2 = jnp.dot(lt, oh2, preferred_element_type=jnp.float32)
    tot1 = jnp.sum(oh1, axis=0, keepdims=True)
    tot2 = jnp.sum(oh2, axis=0, keepdims=True)
    basef = base.astype(jnp.float32)
    rank1 = jnp.sum(jnp.where(oh1 > 0, c1 + basef, 0.0), axis=1, keepdims=True)
    rank2 = jnp.sum(jnp.where(oh2 > 0, c2 + basef + tot1, 0.0), axis=1, keepdims=True)
    r0_r[...] = rank1.astype(jnp.int32)
    r1_r[...] = rank2.astype(jnp.int32)
    newc = base + (tot1 + tot2).astype(jnp.int32)
    cnt_r[...] = jnp.concatenate([newc, jnp.zeros((1, 16 - E), jnp.int32)], axis=1)


def _k1(x2, gate_W, gate_b2, sWg, sbg2, sWu, sbu2, sWd, sbd2):
    return pl.pallas_call(
        _k1_body,
        grid=(NT_TOK,),
        in_specs=[
            pl.BlockSpec((TM, H), lambda i: (i, 0)),
            pl.BlockSpec((H, E), lambda i: (0, 0)),
            pl.BlockSpec((1, E), lambda i: (0, 0)),
            pl.BlockSpec((H, I), lambda i: (0, 0)),
            pl.BlockSpec((1, I), lambda i: (0, 0)),
            pl.BlockSpec((H, I), lambda i: (0, 0)),
            pl.BlockSpec((1, I), lambda i: (0, 0)),
            pl.BlockSpec((I, H), lambda i: (0, 0)),
            pl.BlockSpec((1, H), lambda i: (0, 0)),
        ],
        out_specs=[
            pl.BlockSpec((TM, H), lambda i: (i, 0)),
            pl.BlockSpec((TM, 16), lambda i: (i, 0)),
            pl.BlockSpec((TM, 16), lambda i: (i, 0)),
            pl.BlockSpec((TM, 1), lambda i: (i, 0)),
            pl.BlockSpec((TM, 1), lambda i: (i, 0)),
            pl.BlockSpec((TM, 1), lambda i: (i, 0)),
            pl.BlockSpec((TM, 1), lambda i: (i, 0)),
            pl.BlockSpec((1, 16), lambda i: (0, 0)),
        ],
        out_shape=[
            jax.ShapeDtypeStruct((N, H), jnp.float32),
            jax.ShapeDtypeStruct((N, 16), jnp.float32),
            jax.ShapeDtypeStruct((N, 16), jnp.float32),
            jax.ShapeDtypeStruct((N, 1), jnp.int32),
            jax.ShapeDtypeStruct((N, 1), jnp.int32),
            jax.ShapeDtypeStruct((N, 1), jnp.int32),
            jax.ShapeDtypeStruct((N, 1), jnp.int32),
            jax.ShapeDtypeStruct((1, 16), jnp.int32),
        ],
    )(x2, gate_W, gate_b2, sWg, sbg2, sWu, sbu2, sWd, sbd2)


@functools.partial(
    pl.kernel,
    out_type=[
        jax.ShapeDtypeStruct((R, H), jnp.float32),
        jax.ShapeDtypeStruct((2, N), jnp.int32),
        jax.ShapeDtypeStruct((T_ROWS,), jnp.int32),
        jax.ShapeDtypeStruct((T_ROWS,), jnp.int32),
        jax.ShapeDtypeStruct((T_ROWS,), jnp.int32),
    ],
    mesh=plsc.VectorSubcoreMesh(
        core_axis_name="c", subcore_axis_name="s", num_cores=NC,
        num_subcores=NS),
    compiler_params=pltpu.CompilerParams(needs_layout_passes=False),
    scratch_types=[
        pltpu.VMEM((16,), jnp.int32),
        pltpu.VMEM((SUB,), jnp.int32),
        pltpu.VMEM((SUB,), jnp.int32),
        pltpu.VMEM((SUB,), jnp.int32),
        pltpu.VMEM((SUB,), jnp.int32),
        pltpu.VMEM((SUB, H), jnp.float32),
        pltpu.VMEM((T_ROWS,), jnp.int32),
        pltpu.VMEM((T_ROWS,), jnp.int32),
        pltpu.VMEM((T_ROWS,), jnp.int32),
        pltpu.SemaphoreType.DMA,
        pltpu.SemaphoreType.DMA,
    ],
)
def _dispatch(x_hbm, e0_hbm, e1_hbm, r0_hbm, r1_hbm, cnt_hbm,
              xs_hbm, sl_hbm, sel_hbm, eid_hbm, val_hbm,
              cnt_v, ebuf, rbuf, idx0, idx1, xrows,
              msel, meid, mval, sem0, sem1):
    wid = lax.axis_index("s") * NC + lax.axis_index("c")
    base = wid * CW
    pltpu.sync_copy(cnt_hbm, cnt_v)
    c16 = cnt_v[...]
    tpe = jnp.right_shift(c16 + (TMR - 1), TMR_LOG)
    lane = lax.iota(jnp.int32, 16)
    excl = jnp.zeros((16,), jnp.int32)
    for e in range(E - 1):
        ce = _dg16(tpe, jnp.full((16,), e, jnp.int32))
        excl = excl + jnp.where(lane > e, ce, 0)
    toff = excl * TMR
    cum = excl + tpe
    for j in range(NSUB):
        sb = base + j * SUB
        pltpu.sync_copy(e0_hbm.at[pl.ds(sb, SUB)], ebuf)
        pltpu.sync_copy(r0_hbm.at[pl.ds(sb, SUB)], rbuf)
        for cc in range(SUB // 16):
            sl = pl.ds(cc * 16, 16)
            idx0[sl] = _dg16(toff, ebuf[sl]) + rbuf[sl]
        pltpu.sync_copy(e1_hbm.at[pl.ds(sb, SUB)], ebuf)
        pltpu.sync_copy(r1_hbm.at[pl.ds(sb, SUB)], rbuf)
        for cc in range(SUB // 16):
            sl = pl.ds(cc * 16, 16)
            idx1[sl] = _dg16(toff, ebuf[sl]) + rbuf[sl]
        pltpu.sync_copy(x_hbm.at[pl.ds(sb, SUB)], xrows)
        cp0 = pltpu.async_copy(xrows, xs_hbm.at[idx0], sem0)
        cp1 = pltpu.async_copy(xrows, xs_hbm.at[idx1], sem1)
        pltpu.sync_copy(idx0, sl_hbm.at[0, pl.ds(sb, SUB)])
        pltpu.sync_copy(idx1, sl_hbm.at[1, pl.ds(sb, SUB)])
        cp0.wait()
        cp1.wait()

    @pl.when(wid == 0)
    def _():
        total = _dg16(cum, jnp.full((16,), E - 1, jnp.int32))
        for off in (*range(0, T_ROWS - 16, 16), T_ROWS - 16):
            jv = lax.iota(jnp.int32, 16) + off
            acc = jnp.zeros((16,), jnp.int32)
            for e in range(E):
                ce = _dg16(cum, jnp.full((16,), e, jnp.int32))
                acc = acc + (jv >= ce).astype(jnp.int32)
            vv = (jv < total).astype(jnp.int32)
            sl = pl.ds(off, 16)
            mval[sl] = vv
            meid[sl] = jnp.where(vv > 0, jnp.minimum(acc, E - 1), 0)
            msel[sl] = jnp.where(vv > 0, jv, 0)
        pltpu.sync_copy(msel, sel_hbm)
        pltpu.sync_copy(meid, eid_hbm)
        pltpu.sync_copy(mval, val_hbm)


def _k4_body(sel_r, eid_r, val_r, xs_r, eWg_r, ebg_r, eWu_r, ebu_r, eWd_r, ebd_r,
             ys_r):
    i = pl.program_id(0)

    @pl.when(val_r[i] > 0)
    def _():
        xb = xs_r[...]
        hg = jnp.dot(xb, eWg_r[0], preferred_element_type=jnp.float32) + ebg_r[0]
        hu = jnp.dot(xb, eWu_r[0], preferred_element_type=jnp.float32) + ebu_r[0]
        h = jax.nn.silu(hg) * hu
        ys_r[...] = jnp.dot(h, eWd_r[0], preferred_element_type=jnp.float32) + ebd_r[0]


def _k4(sel, eid, valid, xs, eWg, ebg, eWu, ebu, eWd, ebd):
    grid_spec = pltpu.PrefetchScalarGridSpec(
        num_scalar_prefetch=3,
        grid=(T_ROWS,),
        in_specs=[
            pl.BlockSpec((TMR, H), lambda i, sel, eid, val: (sel[i], 0)),
            pl.BlockSpec((1, H, I), lambda i, sel, eid, val: (eid[i], 0, 0)),
            pl.BlockSpec((1, 1, I), lambda i, sel, eid, val: (eid[i], 0, 0)),
            pl.BlockSpec((1, H, I), lambda i, sel, eid, val: (eid[i], 0, 0)),
            pl.BlockSpec((1, 1, I), lambda i, sel, eid, val: (eid[i], 0, 0)),
            pl.BlockSpec((1, I, H), lambda i, sel, eid, val: (eid[i], 0, 0)),
            pl.BlockSpec((1, 1, H), lambda i, sel, eid, val: (eid[i], 0, 0)),
        ],
        out_specs=pl.BlockSpec((TMR, H), lambda i, sel, eid, val: (i, 0)),
    )
    return pl.pallas_call(
        _k4_body,
        grid_spec=grid_spec,
        out_shape=jax.ShapeDtypeStruct((R, H), jnp.float32),
    )(sel, eid, valid, xs, eWg, ebg, eWu, ebu, eWd, ebd)


def kernel(x, gate_W, gate_b, sWg, sbg, sWu, sbu, sWd, sbd,
           eWg, ebg, eWu, ebu, eWd, ebd):
    x2 = x.reshape(N, H)
    out0, w0b, w1b, e0, e1, r0, r1, cnt = _k1(
        x2, gate_W, gate_b.reshape(1, E),
        sWg, sbg.reshape(1, I), sWu, sbu.reshape(1, I), sWd, sbd.reshape(1, H))
    xs, sl01, sel, eid, val = _dispatch(
        x2, e0.reshape(N), e1.reshape(N), r0.reshape(N), r1.reshape(N),
        cnt.reshape(16))
    ys = _k4(sel, eid, val, xs,
             eWg, ebg.reshape(E, 1, I), eWu, ebu.reshape(E, 1, I),
             eWd, ebd.reshape(E, 1, H))
    g0 = ys[sl01[0]]
    g1 = ys[sl01[1]]
    out = out0 + w0b[:, :1] * g0 + w1b[:, :1] * g1
    return out.reshape(B, S, H)


# speedup vs baseline: 1.0876x; 1.0876x over previous
import functools

import jax
import jax.numpy as jnp
from jax import lax
from jax.experimental import pallas as pl
from jax.experimental.pallas import tpu as pltpu
from jax.experimental.pallas import tpu_sc as plsc

B, S, H, I, E, K = 2, 2048, 1024, 512, 8, 2
N = B * S
TM = 256
TMR = 256
TMR_LOG = 8
NT_TOK = N // TM
T_ROWS = (N * K) // TMR + E
R = T_ROWS * TMR

NC, NS = 2, 16
NW = NC * NS
CW = N // NW
SUB = 64
NSUB = CW // SUB


def _dg16(vals, idx):
    return lax.gather(
        vals, idx[:, None],
        dimension_numbers=lax.GatherDimensionNumbers(
            offset_dims=(), collapsed_slice_dims=(0,), start_index_map=(0,)),
        slice_sizes=(1,),
        mode=lax.GatherScatterMode.PROMISE_IN_BOUNDS)


def _k1_body(x_r, gW_r, gb_r, sWg_r, sbg_r, sWu_r, sbu_r, sWd_r, sbd_r,
             out0_r, w0b_r, w1b_r, e0_r, e1_r, r0_r, r1_r, cnt_r):
    i = pl.program_id(0)
    xb = x_r[...]
    hg = jnp.dot(xb, sWg_r[...], preferred_element_type=jnp.float32) + sbg_r[...]
    hu = jnp.dot(xb, sWu_r[...], preferred_element_type=jnp.float32) + sbu_r[...]
    h = jax.nn.silu(hg) * hu
    out0_r[...] = xb + jnp.dot(h, sWd_r[...], preferred_element_type=jnp.float32) + sbd_r[...]
    logits = jnp.dot(xb, gW_r[...], preferred_element_type=jnp.float32) + gb_r[...]
    p = jax.nn.sigmoid(logits)
    iota_e = lax.broadcasted_iota(jnp.int32, (TM, E), 1)
    v1 = jnp.max(p, axis=1, keepdims=True)
    i1 = jnp.min(jnp.where(p == v1, iota_e, E), axis=1, keepdims=True)
    p2 = jnp.where(iota_e == i1, -1.0, p)
    v2 = jnp.max(p2, axis=1, keepdims=True)
    i2 = jnp.min(jnp.where(p2 == v2, iota_e, E), axis=1, keepdims=True)
    ssum = v1 + v2
    w0b_r[...] = jnp.broadcast_to(v1 / ssum, (TM, 16))
    w1b_r[...] = jnp.broadcast_to(v2 / ssum, (TM, 16))
    e0_r[...] = i1
    e1_r[...] = i2
    @pl.when(i == 0)
    def _():
        cnt_r[...] = jnp.zeros((1, 16), jnp.int32)
    base = cnt_r[:, :E]
    oh1 = (iota_e == i1).astype(jnp.float32)
    oh2 = (iota_e == i2).astype(jnp.float32)
    r_io = lax.broadcasted_iota(jnp.int32, (TM, TM), 0)
    c_io = lax.broadcasted_iota(jnp.int32, (TM, TM), 1)
    lt = (r_io > c_io).astype(jnp.float32)
    c1 = jnp.dot(lt, oh1, preferred_element_type=jnp.float32)
    c2 = jnp.dot(lt, oh2, preferred_element_type=jnp.float32)
    tot1 = jnp.sum(oh1, axis=0, keepdims=True)
    tot2 = jnp.sum(oh2, axis=0, keepdims=True)
    basef = base.astype(jnp.float32)
    rank1 = jnp.sum(jnp.where(oh1 > 0, c1 + basef, 0.0), axis=1, keepdims=True)
    rank2 = jnp.sum(jnp.where(oh2 > 0, c2 + basef + tot1, 0.0), axis=1, keepdims=True)
    r0_r[...] = rank1.astype(jnp.int32)
    r1_r[...] = rank2.astype(jnp.int32)
    newc = base + (tot1 + tot2).astype(jnp.int32)
    cnt_r[...] = jnp.concatenate([newc, jnp.zeros((1, 16 - E), jnp.int32)], axis=1)


def _k1(x2, gate_W, gate_b2, sWg, sbg2, sWu, sbu2, sWd, sbd2):
    return pl.pallas_call(
        _k1_body,
        grid=(NT_TOK,),
        in_specs=[
            pl.BlockSpec((TM, H), lambda i: (i, 0)),
            pl.BlockSpec((H, E), lambda i: (0, 0)),
            pl.BlockSpec((1, E), lambda i: (0, 0)),
            pl.BlockSpec((H, I), lambda i: (0, 0)),
            pl.BlockSpec((1, I), lambda i: (0, 0)),
            pl.BlockSpec((H, I), lambda i: (0, 0)),
            pl.BlockSpec((1, I), lambda i: (0, 0)),
            pl.BlockSpec((I, H), lambda i: (0, 0)),
            pl.BlockSpec((1, H), lambda i: (0, 0)),
        ],
        out_specs=[
            pl.BlockSpec((TM, H), lambda i: (i, 0)),
            pl.BlockSpec((TM, 16), lambda i: (i, 0)),
            pl.BlockSpec((TM, 16), lambda i: (i, 0)),
            pl.BlockSpec((TM, 1), lambda i: (i, 0)),
            pl.BlockSpec((TM, 1), lambda i: (i, 0)),
            pl.BlockSpec((TM, 1), lambda i: (i, 0)),
            pl.BlockSpec((TM, 1), lambda i: (i, 0)),
            pl.BlockSpec((1, 16), lambda i: (0, 0)),
        ],
        out_shape=[
            jax.ShapeDtypeStruct((N, H), jnp.float32),
            jax.ShapeDtypeStruct((N, 16), jnp.float32),
            jax.ShapeDtypeStruct((N, 16), jnp.float32),
            jax.ShapeDtypeStruct((N, 1), jnp.int32),
            jax.ShapeDtypeStruct((N, 1), jnp.int32),
            jax.ShapeDtypeStruct((N, 1), jnp.int32),
            jax.ShapeDtypeStruct((N, 1), jnp.int32),
            jax.ShapeDtypeStruct((1, 16), jnp.int32),
        ],
    )(x2, gate_W, gate_b2, sWg, sbg2, sWu, sbu2, sWd, sbd2)


@functools.partial(
    pl.kernel,
    out_type=[
        jax.ShapeDtypeStruct((R, H), jnp.float32),
        jax.ShapeDtypeStruct((2, N), jnp.int32),
        jax.ShapeDtypeStruct((T_ROWS,), jnp.int32),
        jax.ShapeDtypeStruct((T_ROWS,), jnp.int32),
        jax.ShapeDtypeStruct((T_ROWS,), jnp.int32),
    ],
    mesh=plsc.VectorSubcoreMesh(
        core_axis_name="c", subcore_axis_name="s", num_cores=NC,
        num_subcores=NS),
    compiler_params=pltpu.CompilerParams(needs_layout_passes=False),
    scratch_types=[
        pltpu.VMEM((16,), jnp.int32),
        pltpu.VMEM((SUB,), jnp.int32),
        pltpu.VMEM((SUB,), jnp.int32),
        pltpu.VMEM((SUB,), jnp.int32),
        pltpu.VMEM((SUB,), jnp.int32),
        pltpu.VMEM((SUB, H), jnp.float32),
        pltpu.VMEM((T_ROWS,), jnp.int32),
        pltpu.VMEM((T_ROWS,), jnp.int32),
        pltpu.VMEM((T_ROWS,), jnp.int32),
        pltpu.SemaphoreType.DMA,
        pltpu.SemaphoreType.DMA,
    ],
)
def _dispatch(x_hbm, e0_hbm, e1_hbm, r0_hbm, r1_hbm, cnt_hbm,
              xs_hbm, sl_hbm, sel_hbm, eid_hbm, val_hbm,
              cnt_v, ebuf, rbuf, idx0, idx1, xrows,
              msel, meid, mval, sem0, sem1):
    wid = lax.axis_index("s") * NC + lax.axis_index("c")
    base = wid * CW
    pltpu.sync_copy(cnt_hbm, cnt_v)
    c16 = cnt_v[...]
    tpe = jnp.right_shift(c16 + (TMR - 1), TMR_LOG)
    lane = lax.iota(jnp.int32, 16)
    excl = jnp.zeros((16,), jnp.int32)
    for e in range(E - 1):
        ce = _dg16(tpe, jnp.full((16,), e, jnp.int32))
        excl = excl + jnp.where(lane > e, ce, 0)
    toff = excl * TMR
    cum = excl + tpe
    for j in range(NSUB):
        sb = base + j * SUB
        pltpu.sync_copy(e0_hbm.at[pl.ds(sb, SUB)], ebuf)
        pltpu.sync_copy(r0_hbm.at[pl.ds(sb, SUB)], rbuf)
        for cc in range(SUB // 16):
            sl = pl.ds(cc * 16, 16)
            idx0[sl] = _dg16(toff, ebuf[sl]) + rbuf[sl]
        pltpu.sync_copy(e1_hbm.at[pl.ds(sb, SUB)], ebuf)
        pltpu.sync_copy(r1_hbm.at[pl.ds(sb, SUB)], rbuf)
        for cc in range(SUB // 16):
            sl = pl.ds(cc * 16, 16)
            idx1[sl] = _dg16(toff, ebuf[sl]) + rbuf[sl]
        pltpu.sync_copy(x_hbm.at[pl.ds(sb, SUB)], xrows)
        cp0 = pltpu.async_copy(xrows, xs_hbm.at[idx0], sem0)
        cp1 = pltpu.async_copy(xrows, xs_hbm.at[idx1], sem1)
        pltpu.sync_copy(idx0, sl_hbm.at[0, pl.ds(sb, SUB)])
        pltpu.sync_copy(idx1, sl_hbm.at[1, pl.ds(sb, SUB)])
        cp0.wait()
        cp1.wait()

    @pl.when(wid == 0)
    def _():
        total = _dg16(cum, jnp.full((16,), E - 1, jnp.int32))
        for off in (*range(0, T_ROWS - 16, 16), T_ROWS - 16):
            jv = lax.iota(jnp.int32, 16) + off
            acc = jnp.zeros((16,), jnp.int32)
            for e in range(E):
                ce = _dg16(cum, jnp.full((16,), e, jnp.int32))
                acc = acc + (jv >= ce).astype(jnp.int32)
            vv = (jv < total).astype(jnp.int32)
            sl = pl.ds(off, 16)
            mval[sl] = vv
            meid[sl] = jnp.where(vv > 0, jnp.minimum(acc, E - 1), 0)
            msel[sl] = jnp.where(vv > 0, jv, 0)
        pltpu.sync_copy(msel, sel_hbm)
        pltpu.sync_copy(meid, eid_hbm)
        pltpu.sync_copy(mval, val_hbm)


def _k4_body(sel_r, eid_r, val_r, xs_r, eWg_r, ebg_r, eWu_r, ebu_r, eWd_r, ebd_r,
             ys_r):
    i = pl.program_id(0)

    @pl.when(val_r[i] > 0)
    def _():
        xb = xs_r[...]
        hg = jnp.dot(xb, eWg_r[0], preferred_element_type=jnp.float32) + ebg_r[0]
        hu = jnp.dot(xb, eWu_r[0], preferred_element_type=jnp.float32) + ebu_r[0]
        h = jax.nn.silu(hg) * hu
        ys_r[...] = jnp.dot(h, eWd_r[0], preferred_element_type=jnp.float32) + ebd_r[0]


def _k4(sel, eid, valid, xs, eWg, ebg, eWu, ebu, eWd, ebd):
    grid_spec = pltpu.PrefetchScalarGridSpec(
        num_scalar_prefetch=3,
        grid=(T_ROWS,),
        in_specs=[
            pl.BlockSpec((TMR, H), lambda i, sel, eid, val: (sel[i], 0)),
            pl.BlockSpec((1, H, I), lambda i, sel, eid, val: (eid[i], 0, 0)),
            pl.BlockSpec((1, 1, I), lambda i, sel, eid, val: (eid[i], 0, 0)),
            pl.BlockSpec((1, H, I), lambda i, sel, eid, val: (eid[i], 0, 0)),
            pl.BlockSpec((1, 1, I), lambda i, sel, eid, val: (eid[i], 0, 0)),
            pl.BlockSpec((1, I, H), lambda i, sel, eid, val: (eid[i], 0, 0)),
            pl.BlockSpec((1, 1, H), lambda i, sel, eid, val: (eid[i], 0, 0)),
        ],
        out_specs=pl.BlockSpec((TMR, H), lambda i, sel, eid, val: (i, 0)),
    )
    return pl.pallas_call(
        _k4_body,
        grid_spec=grid_spec,
        out_shape=jax.ShapeDtypeStruct((R, H), jnp.float32),
    )(sel, eid, valid, xs, eWg, ebg, eWu, ebu, eWd, ebd)


def kernel(x, gate_W, gate_b, sWg, sbg, sWu, sbu, sWd, sbd,
           eWg, ebg, eWu, ebu, eWd, ebd):
    x2 = x.reshape(N, H)
    out0, w0b, w1b, e0, e1, r0, r1, cnt = _k1(
        x2, gate_W, gate_b.reshape(1, E),
        sWg, sbg.reshape(1, I), sWu, sbu.reshape(1, I), sWd, sbd.reshape(1, H))
    xs, sl01, sel, eid, val = _dispatch(
        x2, e0.reshape(N), e1.reshape(N), r0.reshape(N), r1.reshape(N),
        cnt.reshape(16))
    ys = _k4(sel, eid, val, xs,
             eWg, ebg.reshape(E, 1, I), eWu, ebu.reshape(E, 1, I),
             eWd, ebd.reshape(E, 1, H))
    g0 = ys[sl01[0]]
    g1 = ys[sl01[1]]
    out = out0 + w0b[:, :1] * g0 + w1b[:, :1] * g1
    return out.reshape(B, S, H)


# speedup vs baseline: 1.1421x; 1.0501x over previous
import functools

import jax
import jax.numpy as jnp
from jax import lax
from jax.experimental import pallas as pl
from jax.experimental.pallas import tpu as pltpu
from jax.experimental.pallas import tpu_sc as plsc

B, S, H, I, E, K = 2, 2048, 1024, 512, 8, 2
N = B * S
TM = 256
TMR = 256
TMR_LOG = 8
NT_TOK = N // TM
T_ROWS = (N * K) // TMR + E
R = T_ROWS * TMR

NC, NS = 2, 16
NW = NC * NS
CW = N // NW
SUB = 64
NSUB = CW // SUB


def _dg16(vals, idx):
    return lax.gather(
        vals, idx[:, None],
        dimension_numbers=lax.GatherDimensionNumbers(
            offset_dims=(), collapsed_slice_dims=(0,), start_index_map=(0,)),
        slice_sizes=(1,),
        mode=lax.GatherScatterMode.PROMISE_IN_BOUNDS)


def _k1b_body(x_r, sWg_r, sbg_r, sWu_r, sbu_r, sWd_r, sbd_r, out0_r):
    xb = x_r[...]
    hg = jnp.dot(xb, sWg_r[...], preferred_element_type=jnp.float32) + sbg_r[...]
    hu = jnp.dot(xb, sWu_r[...], preferred_element_type=jnp.float32) + sbu_r[...]
    h = jax.nn.silu(hg) * hu
    out0_r[...] = xb + jnp.dot(h, sWd_r[...], preferred_element_type=jnp.float32) + sbd_r[...]


def _k1b(x2, sWg, sbg2, sWu, sbu2, sWd, sbd2):
    return pl.pallas_call(
        _k1b_body,
        grid=(NT_TOK,),
        in_specs=[
            pl.BlockSpec((TM, H), lambda i: (i, 0)),
            pl.BlockSpec((H, I), lambda i: (0, 0)),
            pl.BlockSpec((1, I), lambda i: (0, 0)),
            pl.BlockSpec((H, I), lambda i: (0, 0)),
            pl.BlockSpec((1, I), lambda i: (0, 0)),
            pl.BlockSpec((I, H), lambda i: (0, 0)),
            pl.BlockSpec((1, H), lambda i: (0, 0)),
        ],
        out_specs=pl.BlockSpec((TM, H), lambda i: (i, 0)),
        out_shape=jax.ShapeDtypeStruct((N, H), jnp.float32),
    )(x2, sWg, sbg2, sWu, sbu2, sWd, sbd2)


def _k1_body(x_r, gW_r, gb_r,
             w0b_r, w1b_r, e0_r, e1_r, r0_r, r1_r, cnt_r):
    i = pl.program_id(0)
    xb = x_r[...]
    logits = jnp.dot(xb, gW_r[...], preferred_element_type=jnp.float32) + gb_r[...]
    p = jax.nn.sigmoid(logits)
    iota_e = lax.broadcasted_iota(jnp.int32, (TM, E), 1)
    v1 = jnp.max(p, axis=1, keepdims=True)
    i1 = jnp.min(jnp.where(p == v1, iota_e, E), axis=1, keepdims=True)
    p2 = jnp.where(iota_e == i1, -1.0, p)
    v2 = jnp.max(p2, axis=1, keepdims=True)
    i2 = jnp.min(jnp.where(p2 == v2, iota_e, E), axis=1, keepdims=True)
    ssum = v1 + v2
    w0b_r[...] = jnp.broadcast_to(v1 / ssum, (TM, 16))
    w1b_r[...] = jnp.broadcast_to(v2 / ssum, (TM, 16))
    e0_r[...] = i1
    e1_r[...] = i2
    @pl.when(i == 0)
    def _():
        cnt_r[...] = jnp.zeros((1, 16), jnp.int32)
    base = cnt_r[:, :E]
    oh1 = (iota_e == i1).astype(jnp.float32)
    oh2 = (iota_e == i2).astype(jnp.float32)
    r_io = lax.broadcasted_iota(jnp.int32, (TM, TM), 0)
    c_io = lax.broadcasted_iota(jnp.int32, (TM, TM), 1)
    lt = (r_io > c_io).astype(jnp.float32)
    c1 = jnp.dot(lt, oh1, preferred_element_type=jnp.float32)
    c2 = jnp.dot(lt, oh2, preferred_element_type=jnp.float32)
    tot1 = jnp.sum(oh1, axis=0, keepdims=True)
    tot2 = jnp.sum(oh2, axis=0, keepdims=True)
    basef = base.astype(jnp.float32)
    rank1 = jnp.sum(jnp.where(oh1 > 0, c1 + basef, 0.0), axis=1, keepdims=True)
    rank2 = jnp.sum(jnp.where(oh2 > 0, c2 + basef + tot1, 0.0), axis=1, keepdims=True)
    r0_r[...] = rank1.astype(jnp.int32)
    r1_r[...] = rank2.astype(jnp.int32)
    newc = base + (tot1 + tot2).astype(jnp.int32)
    cnt_r[...] = jnp.concatenate([newc, jnp.zeros((1, 16 - E), jnp.int32)], axis=1)


def _k1(x2, gate_W, gate_b2):
    return pl.pallas_call(
        _k1_body,
        grid=(NT_TOK,),
        in_specs=[
            pl.BlockSpec((TM, H), lambda i: (i, 0)),
            pl.BlockSpec((H, E), lambda i: (0, 0)),
            pl.BlockSpec((1, E), lambda i: (0, 0)),
        ],
        out_specs=[
            pl.BlockSpec((TM, 16), lambda i: (i, 0)),
            pl.BlockSpec((TM, 16), lambda i: (i, 0)),
            pl.BlockSpec((TM, 1), lambda i: (i, 0)),
            pl.BlockSpec((TM, 1), lambda i: (i, 0)),
            pl.BlockSpec((TM, 1), lambda i: (i, 0)),
            pl.BlockSpec((TM, 1), lambda i: (i, 0)),
            pl.BlockSpec((1, 16), lambda i: (0, 0)),
        ],
        out_shape=[
            jax.ShapeDtypeStruct((N, 16), jnp.float32),
            jax.ShapeDtypeStruct((N, 16), jnp.float32),
            jax.ShapeDtypeStruct((N, 1), jnp.int32),
            jax.ShapeDtypeStruct((N, 1), jnp.int32),
            jax.ShapeDtypeStruct((N, 1), jnp.int32),
            jax.ShapeDtypeStruct((N, 1), jnp.int32),
            jax.ShapeDtypeStruct((1, 16), jnp.int32),
        ],
    )(x2, gate_W, gate_b2)


@functools.partial(
    pl.kernel,
    out_type=[
        jax.ShapeDtypeStruct((R, H), jnp.float32),
        jax.ShapeDtypeStruct((2, N), jnp.int32),
        jax.ShapeDtypeStruct((T_ROWS,), jnp.int32),
        jax.ShapeDtypeStruct((T_ROWS,), jnp.int32),
        jax.ShapeDtypeStruct((T_ROWS,), jnp.int32),
    ],
    mesh=plsc.VectorSubcoreMesh(
        core_axis_name="c", subcore_axis_name="s", num_cores=NC,
        num_subcores=NS),
    compiler_params=pltpu.CompilerParams(needs_layout_passes=False),
    scratch_types=[
        pltpu.VMEM((16,), jnp.int32),
        pltpu.VMEM((SUB,), jnp.int32),
        pltpu.VMEM((SUB,), jnp.int32),
        pltpu.VMEM((SUB,), jnp.int32),
        pltpu.VMEM((SUB,), jnp.int32),
        pltpu.VMEM((SUB, H), jnp.float32),
        pltpu.VMEM((T_ROWS,), jnp.int32),
        pltpu.VMEM((T_ROWS,), jnp.int32),
        pltpu.VMEM((T_ROWS,), jnp.int32),
        pltpu.SemaphoreType.DMA,
        pltpu.SemaphoreType.DMA,
    ],
)
def _dispatch(x_hbm, e0_hbm, e1_hbm, r0_hbm, r1_hbm, cnt_hbm,
              xs_hbm, sl_hbm, sel_hbm, eid_hbm, val_hbm,
              cnt_v, ebuf, rbuf, idx0, idx1, xrows,
              msel, meid, mval, sem0, sem1):
    wid = lax.axis_index("s") * NC + lax.axis_index("c")
    base = wid * CW
    pltpu.sync_copy(cnt_hbm, cnt_v)
    c16 = cnt_v[...]
    tpe = jnp.right_shift(c16 + (TMR - 1), TMR_LOG)
    lane = lax.iota(jnp.int32, 16)
    excl = jnp.zeros((16,), jnp.int32)
    for e in range(E - 1):
        ce = _dg16(tpe, jnp.full((16,), e, jnp.int32))
        excl = excl + jnp.where(lane > e, ce, 0)
    toff = excl * TMR
    cum = excl + tpe
    for j in range(NSUB):
        sb = base + j * SUB
        pltpu.sync_copy(e0_hbm.at[pl.ds(sb, SUB)], ebuf)
        pltpu.sync_copy(r0_hbm.at[pl.ds(sb, SUB)], rbuf)
        for cc in range(SUB // 16):
            sl = pl.ds(cc * 16, 16)
            idx0[sl] = _dg16(toff, ebuf[sl]) + rbuf[sl]
        pltpu.sync_copy(e1_hbm.at[pl.ds(sb, SUB)], ebuf)
        pltpu.sync_copy(r1_hbm.at[pl.ds(sb, SUB)], rbuf)
        for cc in range(SUB // 16):
            sl = pl.ds(cc * 16, 16)
            idx1[sl] = _dg16(toff, ebuf[sl]) + rbuf[sl]
        pltpu.sync_copy(x_hbm.at[pl.ds(sb, SUB)], xrows)
        cp0 = pltpu.async_copy(xrows, xs_hbm.at[idx0], sem0)
        cp1 = pltpu.async_copy(xrows, xs_hbm.at[idx1], sem1)
        pltpu.sync_copy(idx0, sl_hbm.at[0, pl.ds(sb, SUB)])
        pltpu.sync_copy(idx1, sl_hbm.at[1, pl.ds(sb, SUB)])
        cp0.wait()
        cp1.wait()

    @pl.when(wid == 0)
    def _():
        total = _dg16(cum, jnp.full((16,), E - 1, jnp.int32))
        for off in (*range(0, T_ROWS - 16, 16), T_ROWS - 16):
            jv = lax.iota(jnp.int32, 16) + off
            acc = jnp.zeros((16,), jnp.int32)
            for e in range(E):
                ce = _dg16(cum, jnp.full((16,), e, jnp.int32))
                acc = acc + (jv >= ce).astype(jnp.int32)
            vv = (jv < total).astype(jnp.int32)
            sl = pl.ds(off, 16)
            mval[sl] = vv
            meid[sl] = jnp.where(vv > 0, jnp.minimum(acc, E - 1), 0)
            msel[sl] = jnp.where(vv > 0, jv, 0)
        pltpu.sync_copy(msel, sel_hbm)
        pltpu.sync_copy(meid, eid_hbm)
        pltpu.sync_copy(mval, val_hbm)


def _k4_body(sel_r, eid_r, val_r, xs_r, eWg_r, ebg_r, eWu_r, ebu_r, eWd_r, ebd_r,
             ys_r):
    i = pl.program_id(0)

    @pl.when(val_r[i] > 0)
    def _():
        xb = xs_r[...]
        hg = jnp.dot(xb, eWg_r[0], preferred_element_type=jnp.float32) + ebg_r[0]
        hu = jnp.dot(xb, eWu_r[0], preferred_element_type=jnp.float32) + ebu_r[0]
        h = jax.nn.silu(hg) * hu
        ys_r[...] = jnp.dot(h, eWd_r[0], preferred_element_type=jnp.float32) + ebd_r[0]


def _k4(sel, eid, valid, xs, eWg, ebg, eWu, ebu, eWd, ebd):
    grid_spec = pltpu.PrefetchScalarGridSpec(
        num_scalar_prefetch=3,
        grid=(T_ROWS,),
        in_specs=[
            pl.BlockSpec((TMR, H), lambda i, sel, eid, val: (sel[i], 0)),
            pl.BlockSpec((1, H, I), lambda i, sel, eid, val: (eid[i], 0, 0)),
            pl.BlockSpec((1, 1, I), lambda i, sel, eid, val: (eid[i], 0, 0)),
            pl.BlockSpec((1, H, I), lambda i, sel, eid, val: (eid[i], 0, 0)),
            pl.BlockSpec((1, 1, I), lambda i, sel, eid, val: (eid[i], 0, 0)),
            pl.BlockSpec((1, I, H), lambda i, sel, eid, val: (eid[i], 0, 0)),
            pl.BlockSpec((1, 1, H), lambda i, sel, eid, val: (eid[i], 0, 0)),
        ],
        out_specs=pl.BlockSpec((TMR, H), lambda i, sel, eid, val: (i, 0)),
    )
    return pl.pallas_call(
        _k4_body,
        grid_spec=grid_spec,
        out_shape=jax.ShapeDtypeStruct((R, H), jnp.float32),
    )(sel, eid, valid, xs, eWg, ebg, eWu, ebu, eWd, ebd)


def kernel(x, gate_W, gate_b, sWg, sbg, sWu, sbu, sWd, sbd,
           eWg, ebg, eWu, ebu, eWd, ebd):
    x2 = x.reshape(N, H)
    w0b, w1b, e0, e1, r0, r1, cnt = _k1(x2, gate_W, gate_b.reshape(1, E))
    out0 = _k1b(x2, sWg, sbg.reshape(1, I), sWu, sbu.reshape(1, I),
                sWd, sbd.reshape(1, H))
    xs, sl01, sel, eid, val = _dispatch(
        x2, e0.reshape(N), e1.reshape(N), r0.reshape(N), r1.reshape(N),
        cnt.reshape(16))
    ys = _k4(sel, eid, val, xs,
             eWg, ebg.reshape(E, 1, I), eWu, ebu.reshape(E, 1, I),
             eWd, ebd.reshape(E, 1, H))
    g0 = ys[sl01[0]]
    g1 = ys[sl01[1]]
    out = out0 + w0b[:, :1] * g0 + w1b[:, :1] * g1
    return out.reshape(B, S, H)
